# Initial kernel scaffold; baseline (speedup 1.0000x reference)
import jax
import jax.numpy as jnp
from jax.experimental import pallas as pl


def kernel(node_inputs, edge_index, edge_inputs, W_src, W_dst, b_dst, W_attn_src, W_attn_dst, W_attn_edge):
    raise NotImplementedError("write your pallas kernel here")


# speedup vs baseline: 18.5586x; 18.5586x over previous
import functools

import jax
import jax.numpy as jnp
from jax import lax
from jax.experimental import pallas as pl
from jax.experimental.pallas import tpu as pltpu
from jax.experimental.pallas import tpu_sc as plsc

N = 10000
E = 320000
D = 128
DE = 16
H = 4
F = 32
HF = H * F
RW = HF + 16
NEG = 0.2

NC = 2
NS = 16
NW = NC * NS
EW = E // NW
SB = 80
CPT = EW // SB
SC_CH = 4000
NCH = E // SC_CH
MB = 64
AROWS = 320

_mesh = plsc.VectorSubcoreMesh(core_axis_name="c", subcore_axis_name="s")
_sc_params = pltpu.CompilerParams(needs_layout_passes=False,
                                  use_tc_tiling_on_sc=False)


def _proj_body(x_ref, ws_ref, wa_ref, f_ref, as_ref, ad_ref):
    x = x_ref[...]
    f_ref[...] = jnp.dot(x, ws_ref[...], preferred_element_type=jnp.float32)
    an = jnp.dot(x, wa_ref[...], preferred_element_type=jnp.float32)
    z = jnp.zeros((x.shape[0], 16 - H), jnp.float32)
    as_ref[...] = jnp.concatenate([an[:, :H], z], axis=1)
    ad_ref[...] = jnp.concatenate([an[:, H:], z], axis=1)


def _edge_proj_body(xe_ref, we_ref, ae_ref):
    ae = jnp.dot(xe_ref[...], we_ref[...],
                 preferred_element_type=jnp.float32)
    z = jnp.zeros((ae.shape[0], 16 - H), jnp.float32)
    ae_ref[...] = jnp.concatenate([ae, z], axis=1)


def _rdenom_body(d_ref, r_ref):
    r_ref[...] = 1.0 / jnp.maximum(d_ref[...], 1e-9)


def _combine_body(x_ref, wd_ref, b_ref, p_ref, rde_ref, o_ref):
    fd = jnp.dot(x_ref[...], wd_ref[...], preferred_element_type=jnp.float32)
    o_ref[...] = fd + b_ref[...] + p_ref[...] * rde_ref[...]


@functools.partial(
    pl.kernel,
    out_type=tuple(jax.ShapeDtypeStruct((E,), jnp.float32) for _ in range(H)),
    mesh=_mesh,
    scratch_types=(
        [pltpu.VMEM((N * 2 * H,), jnp.float32)]
        + [pltpu.VMEM((SB,), jnp.int32)]
        + [pltpu.VMEM((SB,), jnp.int32)]
        + [pltpu.VMEM((SB * H,), jnp.float32)]
        + [pltpu.VMEM((H * SB,), jnp.float32)]
        + [pltpu.SemaphoreType.DMA, pltpu.SemaphoreType.DMA]
    ),
    compiler_params=_sc_params,
)
def _sc_logits(src_hbm, dst_hbm, ae_hbm, an_hbm, *refs):
    num_hbm = refs[:H]
    a_tbl, sidx, didx, ae_buf, numf, sem_in, sem_out = refs[H:]

    cid = lax.axis_index("c")
    sid = lax.axis_index("s")
    wid = sid * NC + cid

    pltpu.sync_copy(an_hbm, a_tbl)
    lane = lax.iota(jnp.int32, 16)
    base0 = wid * EW

    def chunk_body(k, carry):
        base = base0 + k * SB
        descs = [
            pltpu.async_copy(src_hbm.at[pl.ds(base, SB)], sidx, sem_in),
            pltpu.async_copy(dst_hbm.at[pl.ds(base, SB)], didx, sem_in),
            pltpu.async_copy(ae_hbm.at[pl.ds(base * H, SB * H)],
                             ae_buf, sem_in),
        ]
        for dsc in descs:
            dsc.wait()

        def group_body(g, c2):
            off = g * 16
            s16 = sidx[pl.ds(off, 16)]
            d16 = didx[pl.ds(off, 16)]
            ig16 = (off + lane) * H
            for h in range(H):
                asrc = plsc.load_gather(a_tbl, [s16 * (2 * H) + h])
                adst = plsc.load_gather(a_tbl, [d16 * (2 * H) + (H + h)])
                ae = plsc.load_gather(ae_buf, [ig16 + h])
                e = asrc + adst + ae
                e = jnp.where(e > 0, e, NEG * e)
                numf[pl.ds(h * SB + off, 16)] = jnp.exp(e)
            return c2

        lax.fori_loop(0, SB // 16, group_body, 0)
        odescs = [pltpu.async_copy(
            numf.at[pl.ds(h * SB, SB)],
            num_hbm[h].at[pl.ds(base, SB)], sem_out) for h in range(H)]
        for dsc in odescs:
            dsc.wait()
        return carry

    lax.fori_loop(0, CPT, chunk_body, 0)


@functools.partial(
    pl.kernel,
    out_type=(jax.ShapeDtypeStruct((E, HF), jnp.float32),
              jax.ShapeDtypeStruct((E, 16), jnp.float32)),
    mesh=_mesh,
    scratch_types=(
        [pltpu.VMEM((SB,), jnp.int32)]
        + [pltpu.VMEM((H * SB + 16,), jnp.float32)]
        + [pltpu.VMEM((SB, HF), jnp.float32)]
        + [pltpu.VMEM((SB, 16), jnp.float32)]
        + [pltpu.SemaphoreType.DMA, pltpu.SemaphoreType.DMA]
    ),
    compiler_params=_sc_params,
)
def _sc_msgs(src_hbm, num0, num1, num2, num3, feat_hbm, m_hbm, md_hbm, *refs):
    num_hbm = (num0, num1, num2, num3)
    sidx, numf, rows, rows_d, sem_in, sem_g = refs

    cid = lax.axis_index("c")
    sid = lax.axis_index("s")
    wid = sid * NC + cid

    numf[pl.ds(H * SB, 16)] = jnp.zeros((16,), jnp.float32)
    lane = lax.iota(jnp.int32, 16)
    zpad = jnp.where(lane < H, lane * SB, H * SB)
    base0 = wid * EW

    def chunk_body(k, carry):
        base = base0 + k * SB
        descs = [pltpu.async_copy(num_hbm[h].at[pl.ds(base, SB)],
                                  numf.at[pl.ds(h * SB, SB)], sem_in)
                 for h in range(H)]
        descs.append(pltpu.async_copy(src_hbm.at[pl.ds(base, SB)],
                                      sidx, sem_in))
        for dsc in descs:
            dsc.wait()
        pltpu.async_copy(feat_hbm.at[sidx], rows, sem_g).wait()

        def scale_body(r, c3):
            rv = jnp.full((16,), r, jnp.int32)
            avs = [plsc.load_gather(numf, [rv + h * SB]) for h in range(H)]
            av_t = plsc.load_gather(numf, [zpad + jnp.where(lane < H, r, 0)])
            rows_d[r, :] = av_t
            for q in range(HF // 16):
                seg = rows[r, pl.ds(q * 16, 16)]
                rows[r, pl.ds(q * 16, 16)] = seg * avs[q // 2]
            return c3

        lax.fori_loop(0, SB, scale_body, 0)
        pltpu.sync_copy(rows, m_hbm.at[pl.ds(base, SB)])
        pltpu.sync_copy(rows_d, md_hbm.at[pl.ds(base, SB)])
        return carry

    lax.fori_loop(0, CPT, chunk_body, 0)


@functools.partial(
    pl.kernel,
    out_type=(jax.ShapeDtypeStruct((N, HF), jnp.float32),
              jax.ShapeDtypeStruct((N, 16), jnp.float32)),
    mesh=_mesh,
    scratch_types=(
        [pltpu.VMEM((AROWS, HF), jnp.float32)]
        + [pltpu.VMEM((AROWS, 16), jnp.float32)]
        + [pltpu.VMEM((SC_CH,), jnp.int32)]
        + [pltpu.VMEM((SC_CH,), jnp.int32)]
        + [pltpu.VMEM((SC_CH + MB,), jnp.int32)]
        + [pltpu.VMEM((MB,), jnp.int32)]
        + [pltpu.VMEM((MB,), jnp.int32)]
        + [pltpu.VMEM((MB, 16), jnp.float32)]
        + [pltpu.VMEM((MB, 16), jnp.float32)]
        + [pltpu.VMEM((MB, 16), jnp.float32)]
        + [pltpu.VMEM((MB, HF), jnp.float32)]
        + [pltpu.VMEM((MB * 16,), jnp.float32)]
        + [pltpu.VMEM((MB + 16,), jnp.int32)]
        + [pltpu.SemaphoreType.DMA, pltpu.SemaphoreType.DMA]
    ),
    compiler_params=_sc_params,
)
def _sc_reduce(dst_hbm, src_hbm, as_hbm, ad_hbm, ae_hbm, feat_hbm,
               x_hbm, xd_hbm, *refs):
    (acc, accd, dstc, srcc, midb, sidxb, didxb, asb, adb, aeb,
     mrows, nmb, dlocb, sem_in, sem_g) = refs

    cid = lax.axis_index("c")
    sid = lax.axis_index("s")
    wid = sid * NC + cid

    lo = 312 * wid + 8 * jnp.minimum(wid, 2)
    mine = jnp.where(wid < 2, 320, 312)
    hi = lo + mine

    zed = jnp.zeros((16,), jnp.float32)

    def zacc_body(r, c):
        for q in range(HF // 16):
            acc[r, pl.ds(q * 16, 16)] = zed
        accd[r, :] = zed
        return c

    lax.fori_loop(0, AROWS, zacc_body, 0)

    def zmid_body(g, c):
        midb[pl.ds(g * 16, 16)] = jnp.zeros((16,), jnp.int32)
        return c

    lax.fori_loop(0, (SC_CH + MB) // 16, zmid_body, 0)

    lane = lax.iota(jnp.int32, 16)

    def chunk_body(kc, carry):
        base_e = kc * SC_CH
        pltpu.sync_copy(dst_hbm.at[pl.ds(base_e, SC_CH)], dstc)
        pltpu.sync_copy(src_hbm.at[pl.ds(base_e, SC_CH)], srcc)

        def scan_body(g, mcnt):
            off = g * 16
            d16 = dstc[pl.ds(off, 16)]
            mask = (d16 >= lo) & (d16 < hi)
            plsc.store_compressed(midb.at[pl.ds(mcnt, 16)],
                                  base_e + off + lane, mask=mask)
            return mcnt + jnp.sum(mask.astype(jnp.int32))

        mcnt = lax.fori_loop(0, SC_CH // 16, scan_body, 0)

        def batch_body(b, c2):
            boff = b * MB
            for g in range(MB // 16):
                mid16 = midb[pl.ds(boff + g * 16, 16)]
                loc16 = jnp.clip(mid16 - base_e, 0, SC_CH - 1)
                s16 = plsc.load_gather(srcc, [loc16])
                d16 = plsc.load_gather(dstc, [loc16])
                sidxb[pl.ds(g * 16, 16)] = s16
                didxb[pl.ds(g * 16, 16)] = d16
                dlocb[pl.ds(g * 16, 16)] = d16 - lo
            descs = [
                pltpu.async_copy(ae_hbm.at[midb.at[pl.ds(boff, MB)]],
                                 aeb, sem_g),
                pltpu.async_copy(as_hbm.at[sidxb], asb, sem_g),
                pltpu.async_copy(ad_hbm.at[didxb], adb, sem_g),
                pltpu.async_copy(feat_hbm.at[sidxb], mrows, sem_g),
            ]
            for dsc in descs:
                dsc.wait()
            nrows = jnp.minimum(mcnt - boff, MB)

            def row_body(r, c3):
                e4 = asb[r, :] + adb[r, :] + aeb[r, :]
                e4 = jnp.where(e4 > 0, e4, NEG * e4)
                nm = jnp.where(lane < H, jnp.exp(e4), 0.0)
                nmb[pl.ds(r * 16, 16)] = nm
                dloc = dlocb[pl.ds(r, 16)][0]
                accd[dloc, :] = accd[dloc, :] + nm
                for q in range(HF // 16):
                    avq = plsc.load_gather(
                        nmb, [jnp.full((16,), r * 16 + q // 2, jnp.int32)])
                    seg = acc[dloc, pl.ds(q * 16, 16)]
                    acc[dloc, pl.ds(q * 16, 16)] = (
                        seg + mrows[r, pl.ds(q * 16, 16)] * avq)
                return c3

            lax.fori_loop(0, nrows, row_body, 0)
            return c2

        lax.fori_loop(0, (mcnt + MB - 1) // MB, batch_body, 0)
        return carry

    lax.fori_loop(0, NCH, chunk_body, 0)

    pltpu.sync_copy(acc.at[pl.ds(0, 312)], x_hbm.at[pl.ds(lo, 312)])
    pltpu.sync_copy(accd.at[pl.ds(0, 312)], xd_hbm.at[pl.ds(lo, 312)])

    @pl.when(wid < 2)
    def _():
        pltpu.sync_copy(acc.at[pl.ds(312, 8)], x_hbm.at[pl.ds(lo + 312, 8)])
        pltpu.sync_copy(accd.at[pl.ds(312, 8)], xd_hbm.at[pl.ds(lo + 312, 8)])


def kernel(node_inputs, edge_index, edge_inputs, W_src, W_dst, b_dst,
           W_attn_src, W_attn_dst, W_attn_edge):
    src = edge_index[0].astype(jnp.int32)
    dst = edge_index[1].astype(jnp.int32)
    wa = jnp.concatenate([W_attn_src, W_attn_dst], axis=1)

    BN = 1000
    feat_src, an_src, an_dst = pl.pallas_call(
        _proj_body,
        grid=(N // BN,),
        in_specs=[
            pl.BlockSpec((BN, D), lambda i: (i, 0)),
            pl.BlockSpec((D, HF), lambda i: (0, 0)),
            pl.BlockSpec((D, 2 * H), lambda i: (0, 0)),
        ],
        out_specs=[
            pl.BlockSpec((BN, HF), lambda i: (i, 0)),
            pl.BlockSpec((BN, 16), lambda i: (i, 0)),
            pl.BlockSpec((BN, 16), lambda i: (i, 0)),
        ],
        out_shape=[
            jax.ShapeDtypeStruct((N, HF), jnp.float32),
            jax.ShapeDtypeStruct((N, 16), jnp.float32),
            jax.ShapeDtypeStruct((N, 16), jnp.float32),
        ],
    )(node_inputs, W_src, wa)

    BE = 4000
    ae16 = pl.pallas_call(
        _edge_proj_body,
        grid=(E // BE,),
        in_specs=[
            pl.BlockSpec((BE, DE), lambda i: (i, 0)),
            pl.BlockSpec((DE, H), lambda i: (0, 0)),
        ],
        out_specs=pl.BlockSpec((BE, 16), lambda i: (i, 0)),
        out_shape=jax.ShapeDtypeStruct((E, 16), jnp.float32),
    )(edge_inputs, W_attn_edge)

    xagg, xagg_d = _sc_reduce(dst, src, an_src, an_dst, ae16, feat_src)

    BD = 2000
    rdenom = pl.pallas_call(
        _rdenom_body,
        grid=(N // BD,),
        in_specs=[pl.BlockSpec((BD, H), lambda i: (i, 0))],
        out_specs=pl.BlockSpec((BD, H), lambda i: (i, 0)),
        out_shape=jax.ShapeDtypeStruct((N, H), jnp.float32),
    )(xagg_d[:, :H])

    rde = jnp.repeat(rdenom, F, axis=1)
    out = pl.pallas_call(
        _combine_body,
        grid=(N // BN,),
        in_specs=[
            pl.BlockSpec((BN, D), lambda i: (i, 0)),
            pl.BlockSpec((D, HF), lambda i: (0, 0)),
            pl.BlockSpec((1, HF), lambda i: (0, 0)),
            pl.BlockSpec((BN, HF), lambda i: (i, 0)),
            pl.BlockSpec((BN, HF), lambda i: (i, 0)),
        ],
        out_specs=pl.BlockSpec((BN, HF), lambda i: (i, 0)),
        out_shape=jax.ShapeDtypeStruct((N, HF), jnp.float32),
    )(node_inputs, W_dst, b_dst.reshape(1, HF), xagg, rde)

    return out.reshape(N, H, F)
